# Initial kernel scaffold; baseline (speedup 1.0000x reference)
import jax
import jax.numpy as jnp
from jax.experimental import pallas as pl


def kernel(indices, tables):
    raise NotImplementedError("write your pallas kernel here")


# speedup vs baseline: 6.5308x; 6.5308x over previous
import functools

import jax
import jax.numpy as jnp
from jax import lax
from jax.experimental import pallas as pl
from jax.experimental.pallas import tpu as pltpu
from jax.experimental.pallas import tpu_sc as plsc

_B, _F, _L, _V, _D = 4096, 26, 20, 100000, 32
_N = _B * _F
_NW = 32
_SEG_PER_W = _N // _NW
_SEG = 64
_CHUNKS = _SEG_PER_W // _SEG
_IDX_PER_CHUNK = _SEG * _L
_GATHER_W = 128
_NGATHER = _IDX_PER_CHUNK // _GATHER_W


def kernel(indices, tables):
    tab = tables.reshape(_F * _V, _D)
    offs = (jnp.arange(_F, dtype=jnp.int32) * _V).reshape(1, _F, 1)
    idx = (indices.astype(jnp.int32) + offs).reshape(_N * _L)

    mesh = plsc.VectorSubcoreMesh(core_axis_name="c", subcore_axis_name="s")

    @functools.partial(
        pl.kernel,
        mesh=mesh,
        compiler_params=pltpu.CompilerParams(use_tc_tiling_on_sc=False),
        out_type=jax.ShapeDtypeStruct((_N, _D), jnp.float32),
        scratch_types=[
            pltpu.VMEM((_IDX_PER_CHUNK,), jnp.int32),
            pltpu.VMEM((_IDX_PER_CHUNK, _D), jnp.float32),
            pltpu.VMEM((_SEG, _D), jnp.float32),
            pltpu.SemaphoreType.DMA,
        ],
    )
    def sc_kernel(tab_hbm, idx_hbm, out_hbm, idx_v, rows_v, out_v, sem):
        wid = lax.axis_index("s") * 2 + lax.axis_index("c")
        seg0 = wid * _SEG_PER_W

        @pl.loop(0, _CHUNKS)
        def _(chunk):
            s_base = seg0 + chunk * _SEG
            pltpu.sync_copy(idx_hbm.at[pl.ds(s_base * _L, _IDX_PER_CHUNK)], idx_v)
            copies = []
            for j in range(_NGATHER):
                sl = pl.ds(j * _GATHER_W, _GATHER_W)
                copies.append(
                    pltpu.async_copy(tab_hbm.at[idx_v.at[sl]], rows_v.at[sl], sem)
                )
            for c in copies:
                c.wait()

            @pl.loop(0, _SEG)
            def _(s):
                base = s * _L
                for c in range(2):
                    csl = pl.ds(c * 16, 16)
                    acc = rows_v[base, csl]
                    for l in range(1, _L):
                        acc = acc + rows_v[base + l, csl]
                    out_v[s, csl] = acc

            pltpu.sync_copy(out_v, out_hbm.at[pl.ds(s_base, _SEG)])

    return sc_kernel(tab, idx).reshape(_B, _F, _D)


# speedup vs baseline: 7.2952x; 1.1170x over previous
import functools

import jax
import jax.numpy as jnp
from jax import lax
from jax.experimental import pallas as pl
from jax.experimental.pallas import tpu as pltpu
from jax.experimental.pallas import tpu_sc as plsc

_B, _F, _L, _V, _D = 4096, 26, 20, 100000, 32
_N = _B * _F
_NW = 32
_SEG_PER_W = _N // _NW
_SEG = 64
_CHUNKS = _SEG_PER_W // _SEG
_IDX_PER_CHUNK = _SEG * _L
_GATHER_W = 128
_NGATHER = _IDX_PER_CHUNK // _GATHER_W


def kernel(indices, tables):
    tab = tables.reshape(_F * _V, _D)
    offs = (jnp.arange(_F, dtype=jnp.int32) * _V).reshape(1, _F, 1)
    idx = (indices.astype(jnp.int32) + offs).reshape(_N * _L)

    mesh = plsc.VectorSubcoreMesh(core_axis_name="c", subcore_axis_name="s")

    @functools.partial(
        pl.kernel,
        mesh=mesh,
        compiler_params=pltpu.CompilerParams(use_tc_tiling_on_sc=False),
        out_type=jax.ShapeDtypeStruct((_N, _D), jnp.float32),
        scratch_types=[
            pltpu.VMEM((_IDX_PER_CHUNK,), jnp.int32),
            pltpu.VMEM((_IDX_PER_CHUNK,), jnp.int32),
            pltpu.VMEM((_IDX_PER_CHUNK, _D), jnp.float32),
            pltpu.VMEM((_IDX_PER_CHUNK, _D), jnp.float32),
            pltpu.VMEM((_SEG, _D), jnp.float32),
            pltpu.VMEM((_SEG, _D), jnp.float32),
            pltpu.SemaphoreType.DMA,
            pltpu.SemaphoreType.DMA,
        ],
    )
    def sc_kernel(tab_hbm, idx_hbm, out_hbm,
                  idx0, idx1, rows0, rows1, out0, out1, sem0, sem1):
        wid = lax.axis_index("s") * 2 + lax.axis_index("c")
        seg0 = wid * _SEG_PER_W

        def fire(chunk, idx_v, rows_v, sem):
            s_base = seg0 + chunk * _SEG
            pltpu.sync_copy(idx_hbm.at[pl.ds(s_base * _L, _IDX_PER_CHUNK)], idx_v)
            for j in range(_NGATHER):
                sl = pl.ds(j * _GATHER_W, _GATHER_W)
                pltpu.async_copy(tab_hbm.at[idx_v.at[sl]], rows_v.at[sl], sem)

        def drain(idx_v, rows_v, sem):
            for j in range(_NGATHER):
                sl = pl.ds(j * _GATHER_W, _GATHER_W)
                pltpu.make_async_copy(
                    tab_hbm.at[idx_v.at[sl]], rows_v.at[sl], sem
                ).wait()

        def acc_store(chunk, rows_v, out_v):
            @pl.loop(0, _SEG)
            def _(s):
                base = s * _L
                for c in range(2):
                    csl = pl.ds(c * 16, 16)
                    acc_a = rows_v[base, csl]
                    acc_b = rows_v[base + 1, csl]
                    for l in range(2, _L, 2):
                        acc_a = acc_a + rows_v[base + l, csl]
                        acc_b = acc_b + rows_v[base + l + 1, csl]
                    out_v[s, csl] = acc_a + acc_b

            s_base = seg0 + chunk * _SEG
            pltpu.sync_copy(out_v, out_hbm.at[pl.ds(s_base, _SEG)])

        fire(0, idx0, rows0, sem0)

        @pl.loop(0, _CHUNKS // 2)
        def _(g):
            c0 = 2 * g
            c1 = c0 + 1
            c2 = jnp.where(c0 + 2 >= _CHUNKS, 0, c0 + 2)
            fire(c1, idx1, rows1, sem1)
            drain(idx0, rows0, sem0)
            acc_store(c0, rows0, out0)
            fire(c2, idx0, rows0, sem0)
            drain(idx1, rows1, sem1)
            acc_store(c1, rows1, out1)

        drain(idx0, rows0, sem0)

    return sc_kernel(tab, idx).reshape(_B, _F, _D)
